# baseline (speedup 1.0000x reference)
import functools

import jax
import jax.numpy as jnp
from jax.experimental import pallas as pl
from jax.experimental.pallas import tpu as pltpu


def _eca_kernel(w_ref, x_ref, o_ref, acc_ref, g_ref, *, k_size, pad, inv_hw,
                ch):
    j = pl.program_id(1)

    @pl.when(j == 0)
    def _pool_half0():
        x = x_ref[...]
        acc_ref[:, :ch] = jnp.sum(x, axis=-1, dtype=jnp.float32)
        o_ref[...] = x

    @pl.when(j == 1)
    def _gate_half0():
        x = x_ref[...]
        acc_ref[:, ch:] = jnp.sum(x, axis=-1, dtype=jnp.float32)
        y = acc_ref[...] * inv_hw
        c = y.shape[-1]
        if pad > 0:
            z = jnp.zeros((y.shape[0], pad), dtype=y.dtype)
            yp = jnp.concatenate([z, y, z], axis=-1)
        else:
            yp = y
        out = w_ref[0] * jax.lax.slice_in_dim(yp, 0, c, axis=-1)
        for t in range(1, k_size):
            out = out + w_ref[t] * jax.lax.slice_in_dim(yp, t, t + c, axis=-1)
        g = jax.nn.sigmoid(out)
        g_ref[...] = g
        o_ref[...] = o_ref[...] * g[:, :ch].astype(o_ref.dtype)[:, :, None]

    @pl.when(j == 2)
    def _gate_half1():
        x = x_ref[...]
        g = g_ref[:, ch:]
        o_ref[...] = x * g.astype(o_ref.dtype)[:, :, None]


def kernel(x, conv_w):
    B, C, H, W = x.shape
    HW = H * W
    k_size = conv_w.shape[-1]
    pad = (k_size - 1) // 2
    inv_hw = 1.0 / HW
    ch = C // 2

    x2 = x.reshape(B, C, HW)
    w_flat = conv_w.reshape(k_size).astype(jnp.float32)

    item = jnp.dtype(x.dtype).itemsize
    per_b = ch * HW * item
    bt = max(1, min(B, (48 * 1024 * 1024) // (4 * per_b)))
    while B % bt:
        bt -= 1
    grid = (B // bt, 3)

    out2 = pl.pallas_call(
        functools.partial(_eca_kernel, k_size=k_size, pad=pad, inv_hw=inv_hw,
                          ch=ch),
        out_shape=jax.ShapeDtypeStruct((B, C, HW), x.dtype),
        grid_spec=pltpu.PrefetchScalarGridSpec(
            num_scalar_prefetch=1,
            grid=grid,
            in_specs=[pl.BlockSpec((bt, ch, HW),
                                   lambda b, j, w: (b, (j + 1) // 2, 0))],
            out_specs=pl.BlockSpec((bt, ch, HW),
                                   lambda b, j, w: (b, j // 2, 0)),
            scratch_shapes=[pltpu.VMEM((bt, C), jnp.float32),
                            pltpu.VMEM((bt, C), jnp.float32)]),
        compiler_params=pltpu.CompilerParams(
            dimension_semantics=("parallel", "arbitrary"),
            vmem_limit_bytes=60 * 1024 * 1024),
    )(w_flat, x2)
    return out2.reshape(B, C, H, W)


# speedup vs baseline: 1.0407x; 1.0407x over previous
import functools

import jax
import jax.numpy as jnp
from jax.experimental import pallas as pl
from jax.experimental.pallas import tpu as pltpu


def _eca_kernel(w_ref, x_ref, o_ref, *, k_size, pad):
    x = x_ref[...]
    y = jnp.sum(x, axis=-1, dtype=jnp.float32)
    c = y.shape[-1]
    if pad > 0:
        z = jnp.zeros((y.shape[0], pad), dtype=y.dtype)
        yp = jnp.concatenate([z, y, z], axis=-1)
    else:
        yp = y
    out = w_ref[0] * jax.lax.slice_in_dim(yp, 0, c, axis=-1)
    for t in range(1, k_size):
        out = out + w_ref[t] * jax.lax.slice_in_dim(yp, t, t + c, axis=-1)
    g = jax.nn.sigmoid(out)
    o_ref[...] = x * g.astype(o_ref.dtype)[:, :, None]


def kernel(x, conv_w):
    B, C, H, W = x.shape
    HW = H * W
    k_size = conv_w.shape[-1]
    pad = (k_size - 1) // 2

    x2 = x.reshape(B, C, HW)
    w_flat = conv_w.reshape(k_size).astype(jnp.float32) * (1.0 / HW)

    item = jnp.dtype(x.dtype).itemsize
    per_b = C * HW * item
    bt = max(1, min(B, (48 * 1024 * 1024) // (4 * per_b)))
    while B % bt:
        bt -= 1
    grid = (B // bt,)

    out2 = pl.pallas_call(
        functools.partial(_eca_kernel, k_size=k_size, pad=pad),
        out_shape=jax.ShapeDtypeStruct((B, C, HW), x.dtype),
        grid_spec=pltpu.PrefetchScalarGridSpec(
            num_scalar_prefetch=1,
            grid=grid,
            in_specs=[pl.BlockSpec((bt, C, HW), lambda b, w: (b, 0, 0))],
            out_specs=pl.BlockSpec((bt, C, HW), lambda b, w: (b, 0, 0))),
        compiler_params=pltpu.CompilerParams(
            dimension_semantics=("parallel",),
            vmem_limit_bytes=60 * 1024 * 1024),
    )(w_flat, x2)
    return out2.reshape(B, C, H, W)


# speedup vs baseline: 1.0477x; 1.0067x over previous
import functools

import jax
import jax.numpy as jnp
from jax.experimental import pallas as pl
from jax.experimental.pallas import tpu as pltpu


def _eca_kernel(w_ref, x_ref, o_ref, *, k_size, pad, inv_hw):
    x = x_ref[...]
    y = jnp.sum(x, axis=-1, dtype=jnp.float32) * inv_hw
    c = y.shape[-1]
    if pad > 0:
        z = jnp.zeros((y.shape[0], pad), dtype=y.dtype)
        yp = jnp.concatenate([z, y, z], axis=-1)
    else:
        yp = y
    out = w_ref[0] * jax.lax.slice_in_dim(yp, 0, c, axis=-1)
    for t in range(1, k_size):
        out = out + w_ref[t] * jax.lax.slice_in_dim(yp, t, t + c, axis=-1)
    g = jax.nn.sigmoid(out)
    o_ref[...] = x * g.astype(o_ref.dtype)[:, :, None]


def kernel(x, conv_w):
    B, C, H, W = x.shape
    HW = H * W
    k_size = conv_w.shape[-1]
    pad = (k_size - 1) // 2
    inv_hw = 1.0 / HW

    x2 = x.reshape(B, C, HW)
    w_flat = conv_w.reshape(k_size).astype(jnp.float32)

    item = jnp.dtype(x.dtype).itemsize
    per_b = C * HW * item
    bt = max(1, min(B, (48 * 1024 * 1024) // (4 * per_b)))
    while B % bt:
        bt -= 1
    grid = (B // bt,)

    out2 = pl.pallas_call(
        functools.partial(_eca_kernel, k_size=k_size, pad=pad, inv_hw=inv_hw),
        out_shape=jax.ShapeDtypeStruct((B, C, HW), x.dtype),
        grid_spec=pltpu.PrefetchScalarGridSpec(
            num_scalar_prefetch=1,
            grid=grid,
            in_specs=[pl.BlockSpec((bt, C, HW), lambda b, w: (b, 0, 0))],
            out_specs=pl.BlockSpec((bt, C, HW), lambda b, w: (b, 0, 0))),
        compiler_params=pltpu.CompilerParams(
            dimension_semantics=("parallel",),
            vmem_limit_bytes=60 * 1024 * 1024),
    )(w_flat, x2)
    return out2.reshape(B, C, H, W)
